# baseline (speedup 1.0000x reference)
import jax
import jax.numpy as jnp
from jax.experimental import pallas as pl
from jax.experimental.pallas import tpu as pltpu

_B, _N, _D, _U = 4, 2048, 128, 128


def _mpnn_body(x_ref, adj_ref, wmsg_ref, wupd_ref, out_ref):
    xb = x_ref[0].astype(jnp.bfloat16)
    a = adj_ref[0]
    wm = wmsg_ref[...].astype(jnp.bfloat16)
    wu = wupd_ref[...].astype(jnp.bfloat16)
    msg = jax.lax.dot(xb, wm, preferred_element_type=jnp.float32)
    upd = jax.lax.dot(xb, wu, preferred_element_type=jnp.float32)
    corner = jnp.sum(a[:8, :128].astype(jnp.float32)) * 0.0
    out_ref[0] = upd + msg + corner


def kernel(x, adj, W_msg, W_upd):
    adj = adj.view(jnp.int32)
    return pl.pallas_call(
        _mpnn_body,
        grid=(_B,),
        in_specs=[
            pl.BlockSpec((1, _N, _D), lambda b: (b, 0, 0)),
            pl.BlockSpec((1, _N, _N // 4), lambda b: (b, 0, 0)),
            pl.BlockSpec((_D, _U), lambda b: (0, 0)),
            pl.BlockSpec((_D, _U), lambda b: (0, 0)),
        ],
        out_specs=pl.BlockSpec((1, _N, _U), lambda b: (b, 0, 0)),
        out_shape=jax.ShapeDtypeStruct((_B, _N, _U), jnp.float32),
    )(x, adj, W_msg, W_upd)


# speedup vs baseline: 6.9568x; 6.9568x over previous
import jax
import jax.numpy as jnp
from jax.experimental import pallas as pl
from jax.experimental.pallas import tpu as pltpu

_B, _N, _D, _U = 4, 2048, 128, 128


def _mpnn_body(x_ref, adj_ref, wmsg_ref, wupd_ref, out_ref):
    xT = x_ref[0].astype(jnp.bfloat16).T
    a = adj_ref[0]
    wmT = wmsg_ref[...].astype(jnp.bfloat16).T
    wuT = wupd_ref[...].astype(jnp.bfloat16).T

    msgT = jax.lax.dot(wmT, xT, preferred_element_type=jnp.float32)

    lhs = jnp.concatenate(
        [msgT.astype(jnp.bfloat16), jnp.ones((16, _N), dtype=jnp.bfloat16)],
        axis=0)
    p = jax.lax.dot(lhs, a.astype(jnp.bfloat16),
                    preferred_element_type=jnp.float32)
    aggT = p[:_U]
    deg = p[_U:_U + 1]

    updT = jax.lax.dot(wuT, xT, preferred_element_type=jnp.float32)

    msgs = jnp.where(deg > 0, aggT / jnp.maximum(deg, 1.0), 0.0)
    out_ref[0] = (updT + msgs).T


def kernel(x, adj, W_msg, W_upd):
    adj = adj.view(jnp.int8)
    return pl.pallas_call(
        _mpnn_body,
        grid=(_B,),
        in_specs=[
            pl.BlockSpec((1, _N, _D), lambda b: (b, 0, 0)),
            pl.BlockSpec((1, _N, _N), lambda b: (b, 0, 0)),
            pl.BlockSpec((_D, _U), lambda b: (0, 0)),
            pl.BlockSpec((_D, _U), lambda b: (0, 0)),
        ],
        out_specs=pl.BlockSpec((1, _N, _U), lambda b: (b, 0, 0)),
        out_shape=jax.ShapeDtypeStruct((_B, _N, _U), jnp.float32),
    )(x, adj, W_msg, W_upd)
